# Initial kernel scaffold; baseline (speedup 1.0000x reference)
import jax
import jax.numpy as jnp
from jax.experimental import pallas as pl


def kernel(tensor, embedding_weight, fc_weight, fc_bias):
    raise NotImplementedError("write your pallas kernel here")


# speedup vs baseline: 5.1572x; 5.1572x over previous
import functools

import jax
import jax.numpy as jnp
from jax import lax
from jax.experimental import pallas as pl
from jax.experimental.pallas import tpu as pltpu, tpu_sc as plsc

_INFO = plsc.get_sparse_core_info()
_NC, _NS = _INFO.num_cores, _INFO.num_subcores
_NW = _NC * _NS

_B, _L, _V, _D, _O = 16384, 200, 10, 8, 4
_N = _B * _L
_PER_W = _N // _NW
_CHUNK = 4096
_STEPS = _PER_W // _CHUNK
_PARAMS = 176


def _sc_body(idx_hbm, params_hbm, out_hbm, params_v, ctab_v, idx_v, out_v):
    wid = lax.axis_index("s") * _NC + lax.axis_index("c")
    pltpu.sync_copy(params_hbm, params_v)

    iota = lax.iota(jnp.int32, 16)
    pos = [iota * 4 + o for o in range(_O)]

    ecols = [params_v[pl.ds(d * 16, 16)] for d in range(_D)]
    w_lo = params_v[pl.ds(128, 16)]
    w_hi = params_v[pl.ds(144, 16)]
    bvec = params_v[pl.ds(160, 16)]
    for o in range(_O):
        wrow = w_lo if o < 2 else w_hi
        acc = jnp.zeros((16,), jnp.float32) + bvec[o]
        for d in range(_D):
            acc = acc + ecols[d] * wrow[(o % 2) * _D + d]
        ctab_v[pl.ds(o * 16, 16)] = acc

    base = wid * _PER_W

    def chunk_body(s, carry):
        off = base + s * _CHUNK
        pltpu.sync_copy(idx_hbm.at[pl.ds(off, _CHUNK)], idx_v)

        def grp(q, c):
            idx16 = idx_v[pl.ds(q * 16, 16)]
            qb = q * 64
            for o in range(_O):
                vals = plsc.load_gather(ctab_v, [idx16 + o * 16])
                plsc.store_scatter(out_v, [pos[o] + qb], vals)
            return c

        lax.fori_loop(0, _CHUNK // 16, grp, 0, unroll=2)
        pltpu.sync_copy(out_v, out_hbm.at[pl.ds(off * 4, _CHUNK * 4)])
        return carry

    lax.fori_loop(0, _STEPS, chunk_body, 0)


@jax.jit
def _sc_lookup(idx, params):
    mesh = plsc.VectorSubcoreMesh(core_axis_name="c", subcore_axis_name="s")
    return pl.kernel(
        _sc_body,
        out_type=jax.ShapeDtypeStruct((_N * _O,), jnp.float32),
        mesh=mesh,
        scratch_types=[
            pltpu.VMEM((_PARAMS,), jnp.float32),
            pltpu.VMEM((64,), jnp.float32),
            pltpu.VMEM((_CHUNK,), jnp.int32),
            pltpu.VMEM((_CHUNK * _O,), jnp.float32),
        ],
        compiler_params=pltpu.CompilerParams(needs_layout_passes=False),
    )(idx, params)


def kernel(tensor, embedding_weight, fc_weight, fc_bias):
    idx = tensor.reshape(-1).astype(jnp.int32)
    embT = jnp.zeros((_D, 16), jnp.float32).at[:, :_V].set(
        embedding_weight.astype(jnp.float32).T
    )
    params = jnp.concatenate(
        [
            embT.reshape(-1),
            fc_weight.astype(jnp.float32).reshape(-1),
            fc_bias.astype(jnp.float32),
            jnp.zeros((12,), jnp.float32),
        ]
    )
    out = _sc_lookup(idx, params)
    return out.reshape(tensor.shape + (_O,))


# speedup vs baseline: 5.4101x; 1.0490x over previous
import functools

import jax
import jax.numpy as jnp
from jax import lax
from jax.experimental import pallas as pl
from jax.experimental.pallas import tpu as pltpu, tpu_sc as plsc

_INFO = plsc.get_sparse_core_info()
_NC, _NS = _INFO.num_cores, _INFO.num_subcores
_NW = _NC * _NS

_B, _L, _V, _D, _O = 16384, 200, 10, 8, 4
_N = _B * _L
_PER_W = _N // _NW
_CHUNK = 4096
_STEPS = _PER_W // _CHUNK
_PARAMS = 176


def _sc_body(idx_hbm, params_hbm, out_hbm, params_v, ctab_v, idx_v, out_v):
    wid = lax.axis_index("s") * _NC + lax.axis_index("c")
    pltpu.sync_copy(params_hbm, params_v)

    iota = lax.iota(jnp.int32, 16)
    pos = [iota * 4 + o for o in range(_O)]

    ecols = [params_v[pl.ds(d * 16, 16)] for d in range(_D)]
    w_lo = params_v[pl.ds(128, 16)]
    w_hi = params_v[pl.ds(144, 16)]
    bvec = params_v[pl.ds(160, 16)]
    for o in range(_O):
        wrow = w_lo if o < 2 else w_hi
        acc = jnp.zeros((16,), jnp.float32) + bvec[o]
        for d in range(_D):
            acc = acc + ecols[d] * wrow[(o % 2) * _D + d]
        ctab_v[pl.ds(o * 16, 16)] = acc

    base = wid * _PER_W

    def chunk_body(s, carry):
        off = base + s * _CHUNK
        pltpu.sync_copy(idx_hbm.at[pl.ds(off, _CHUNK)], idx_v)

        @plsc.parallel_loop(0, _CHUNK // 16, unroll=8)
        def grp(q):
            idx16 = idx_v[pl.ds(q * 16, 16)]
            qb = q * 64
            for o in range(_O):
                vals = plsc.load_gather(ctab_v, [idx16 + o * 16])
                plsc.store_scatter(out_v, [pos[o] + qb], vals)
        pltpu.sync_copy(out_v, out_hbm.at[pl.ds(off * 4, _CHUNK * 4)])
        return carry

    lax.fori_loop(0, _STEPS, chunk_body, 0)


@jax.jit
def _sc_lookup(idx, params):
    mesh = plsc.VectorSubcoreMesh(core_axis_name="c", subcore_axis_name="s")
    return pl.kernel(
        _sc_body,
        out_type=jax.ShapeDtypeStruct((_N * _O,), jnp.float32),
        mesh=mesh,
        scratch_types=[
            pltpu.VMEM((_PARAMS,), jnp.float32),
            pltpu.VMEM((64,), jnp.float32),
            pltpu.VMEM((_CHUNK,), jnp.int32),
            pltpu.VMEM((_CHUNK * _O,), jnp.float32),
        ],
        compiler_params=pltpu.CompilerParams(needs_layout_passes=False),
    )(idx, params)


def kernel(tensor, embedding_weight, fc_weight, fc_bias):
    idx = tensor.reshape(-1).astype(jnp.int32)
    embT = jnp.zeros((_D, 16), jnp.float32).at[:, :_V].set(
        embedding_weight.astype(jnp.float32).T
    )
    params = jnp.concatenate(
        [
            embT.reshape(-1),
            fc_weight.astype(jnp.float32).reshape(-1),
            fc_bias.astype(jnp.float32),
            jnp.zeros((12,), jnp.float32),
        ]
    )
    out = _sc_lookup(idx, params)
    return out.reshape(tensor.shape + (_O,))


# speedup vs baseline: 223.3477x; 41.2833x over previous
import jax
import jax.numpy as jnp
from jax import lax
from jax.experimental import pallas as pl
from jax.experimental.pallas import tpu as pltpu, tpu_sc as plsc

_INFO = plsc.get_sparse_core_info()
_NC, _NS = _INFO.num_cores, _INFO.num_subcores
_NW = _NC * _NS

_B, _L, _V, _D, _O = 16384, 200, 10, 8, 4
_LT, _BT = _L // 8, _B // 128
_BT_PER_W = _BT // _NW
_LT_CHUNK = 5
_NCHUNK = _LT // _LT_CHUNK
_STEPS = _BT_PER_W * _NCHUNK
_ROWS = _LT_CHUNK * 8
_PARAMS = 176


def _sc_body(q_hbm, params_hbm, p_hbm,
             params_v, t2_v, idx_a, idx_b, out_a, out_b,
             in_sem_a, in_sem_b, out_sem_a, out_sem_b):
    wid = lax.axis_index("s") * _NC + lax.axis_index("c")
    bt0 = wid * _BT_PER_W
    pltpu.sync_copy(params_hbm, params_v)

    iota = lax.iota(jnp.int32, 16)
    oi = [o * 16 + iota for o in range(_O)]

    ecols = [params_v[pl.ds(d * 16, 16)] for d in range(_D)]
    w_lo = params_v[pl.ds(128, 16)]
    w_hi = params_v[pl.ds(144, 16)]
    bvec = params_v[pl.ds(160, 16)]
    for o in range(_O):
        wrow = w_lo if o < 2 else w_hi
        c_o = jnp.zeros((16,), jnp.float32) + bvec[o]
        for d in range(_D):
            c_o = c_o + ecols[d] * wrow[(o % 2) * _D + d]
        for i in range(_V):
            t2_v[pl.ds(i * 64 + o * 16, 16)] = jnp.zeros((16,), jnp.float32) + c_o[i]

    def compute(idx_v, out_v):
        @plsc.parallel_loop(0, _ROWS, unroll=2)
        def row(r):
            for g in range(8):
                idx16 = idx_v[r, pl.ds(g * 16, 16)]
                gg = idx16 * 64
                for o in range(_O):
                    out_v[r, o, pl.ds(g * 16, 16)] = \
                        plsc.load_gather(t2_v, [gg + oi[o]])

    bufs = ((idx_a, out_a, in_sem_a, out_sem_a),
            (idx_b, out_b, in_sem_b, out_sem_b))

    def in_copies(step, bu):
        bt = bt0 + step // _NCHUNK
        lt0 = (step % _NCHUNK) * _LT_CHUNK
        return [
            pltpu.make_async_copy(
                q_hbm.at[lt0 + i, bt], bu[0].at[pl.ds(i * 8, 8)], bu[2])
            for i in range(_LT_CHUNK)
        ]

    def out_copy(step, bu):
        bt = bt0 + step // _NCHUNK
        l0 = (step % _NCHUNK) * _ROWS
        return pltpu.make_async_copy(
            bu[1], p_hbm.at[pl.ds(l0, _ROWS), bt], bu[3])

    for cp in in_copies(0, bufs[0]):
        cp.start()
    for cp in in_copies(1, bufs[1]):
        cp.start()

    @pl.loop(0, _STEPS, step=2)
    def chunk_pair(s):
        for b in range(2):
            bu = bufs[b]
            step = s + b
            for cp in in_copies(step, bu):
                cp.wait()

            @pl.when(step >= 2)
            def _():
                out_copy(step - 2, bu).wait()

            compute(bu[0], bu[1])
            out_copy(step, bu).start()

            @pl.when(step + 2 < _STEPS)
            def _():
                for cp in in_copies(step + 2, bu):
                    cp.start()

    out_copy(_STEPS - 2, bufs[0]).wait()
    out_copy(_STEPS - 1, bufs[1]).wait()


@jax.jit
def _sc_lookup(q, params):
    mesh = plsc.VectorSubcoreMesh(core_axis_name="c", subcore_axis_name="s")
    return pl.kernel(
        _sc_body,
        out_type=jax.ShapeDtypeStruct((_L, _BT, _O, 128), jnp.float32),
        mesh=mesh,
        scratch_types=[
            pltpu.VMEM((_PARAMS,), jnp.float32),
            pltpu.VMEM((_V * 64,), jnp.float32),
            pltpu.VMEM((_ROWS, 128), jnp.int32),
            pltpu.VMEM((_ROWS, 128), jnp.int32),
            pltpu.VMEM((_ROWS, _O, 128), jnp.float32),
            pltpu.VMEM((_ROWS, _O, 128), jnp.float32),
            pltpu.SemaphoreType.DMA,
            pltpu.SemaphoreType.DMA,
            pltpu.SemaphoreType.DMA,
            pltpu.SemaphoreType.DMA,
        ],
        compiler_params=pltpu.CompilerParams(needs_layout_passes=False),
    )(q, params)


def kernel(tensor, embedding_weight, fc_weight, fc_bias):
    q = (
        tensor.astype(jnp.int32)
        .reshape(_BT, 128, _LT, 8)
        .transpose(2, 0, 3, 1)
    )
    embT = jnp.zeros((_D, 16), jnp.float32).at[:, :_V].set(
        embedding_weight.astype(jnp.float32).T
    )
    params = jnp.concatenate(
        [
            embT.reshape(-1),
            fc_weight.astype(jnp.float32).reshape(-1),
            fc_bias.astype(jnp.float32),
            jnp.zeros((12,), jnp.float32),
        ]
    )
    p = _sc_lookup(q, params)
    return p.transpose(1, 3, 0, 2).reshape(_B, _L, _O)


# speedup vs baseline: 224.3886x; 1.0047x over previous
import jax
import jax.numpy as jnp
from jax import lax
from jax.experimental import pallas as pl
from jax.experimental.pallas import tpu as pltpu, tpu_sc as plsc

_INFO = plsc.get_sparse_core_info()
_NC, _NS = _INFO.num_cores, _INFO.num_subcores
_NW = _NC * _NS

_B, _L, _V, _D, _O = 16384, 200, 10, 8, 4
_LT, _BT = _L // 8, _B // 128
_BT_PER_W = _BT // _NW
_LT_CHUNK = 5
_NCHUNK = _LT // _LT_CHUNK
_STEPS = _BT_PER_W * _NCHUNK
_ROWS = _LT_CHUNK * 8
_PARAMS = 176


def _sc_body(q_hbm, params_hbm, p_hbm,
             params_v, t2_v, idx_a, idx_b, out_a, out_b,
             in_sem_a, in_sem_b, out_sem_a, out_sem_b):
    wid = lax.axis_index("s") * _NC + lax.axis_index("c")
    bt0 = wid * _BT_PER_W
    pltpu.sync_copy(params_hbm, params_v)

    iota = lax.iota(jnp.int32, 16)
    oi = [o * 16 + iota for o in range(_O)]

    ecols = [params_v[pl.ds(d * 16, 16)] for d in range(_D)]
    w_lo = params_v[pl.ds(128, 16)]
    w_hi = params_v[pl.ds(144, 16)]
    bvec = params_v[pl.ds(160, 16)]
    for o in range(_O):
        wrow = w_lo if o < 2 else w_hi
        c_o = jnp.zeros((16,), jnp.float32) + bvec[o]
        for d in range(_D):
            c_o = c_o + ecols[d] * wrow[(o % 2) * _D + d]
        for i in range(_V):
            t2_v[pl.ds(i * 64 + o * 16, 16)] = jnp.zeros((16,), jnp.float32) + c_o[i]

    def compute(idx_v, out_v):
        @plsc.parallel_loop(0, _ROWS, unroll=2)
        def row(r):
            for g in range(8):
                idx16 = idx_v[r, pl.ds(g * 16, 16)]
                gg = idx16 * 64
                for o in range(_O):
                    out_v[r, o, pl.ds(g * 16, 16)] = \
                        plsc.load_gather(t2_v, [gg + oi[o]])

    bufs = ((idx_a, out_a, in_sem_a, out_sem_a),
            (idx_b, out_b, in_sem_b, out_sem_b))

    def in_copies(step, bu):
        bt = bt0 + step // _NCHUNK
        lt0 = (step % _NCHUNK) * _LT_CHUNK
        return [
            pltpu.make_async_copy(
                q_hbm.at[lt0 + i, bt], bu[0].at[pl.ds(i * 8, 8)], bu[2])
            for i in range(_LT_CHUNK)
        ]

    def out_copy(step, bu):
        bt = bt0 + step // _NCHUNK
        l0 = (step % _NCHUNK) * _ROWS
        return pltpu.make_async_copy(
            bu[1], p_hbm.at[pl.ds(l0, _ROWS), bt], bu[3])

    for cp in in_copies(0, bufs[0]):
        cp.start()
    for cp in in_copies(1, bufs[1]):
        cp.start()

    @pl.loop(0, _STEPS, step=2)
    def chunk_pair(s):
        for b in range(2):
            bu = bufs[b]
            step = s + b
            for cp in in_copies(step, bu):
                cp.wait()

            @pl.when(step >= 2)
            def _():
                out_copy(step - 2, bu).wait()

            compute(bu[0], bu[1])
            out_copy(step, bu).start()

            @pl.when(step + 2 < _STEPS)
            def _():
                for cp in in_copies(step + 2, bu):
                    cp.start()

    out_copy(_STEPS - 2, bufs[0]).wait()
    out_copy(_STEPS - 1, bufs[1]).wait()


@jax.jit
def _sc_lookup(q, params):
    mesh = plsc.VectorSubcoreMesh(core_axis_name="c", subcore_axis_name="s")
    return pl.kernel(
        _sc_body,
        out_type=jax.ShapeDtypeStruct((_L, _BT, _O, 128), jnp.float32),
        mesh=mesh,
        scratch_types=[
            pltpu.VMEM((_PARAMS,), jnp.float32),
            pltpu.VMEM((_V * 64,), jnp.float32),
            pltpu.VMEM((_ROWS, 128), jnp.int32),
            pltpu.VMEM((_ROWS, 128), jnp.int32),
            pltpu.VMEM((_ROWS, _O, 128), jnp.float32),
            pltpu.VMEM((_ROWS, _O, 128), jnp.float32),
            pltpu.SemaphoreType.DMA,
            pltpu.SemaphoreType.DMA,
            pltpu.SemaphoreType.DMA,
            pltpu.SemaphoreType.DMA,
        ],
        compiler_params=pltpu.CompilerParams(
            needs_layout_passes=False, skip_device_barrier=True
        ),
    )(q, params)


def kernel(tensor, embedding_weight, fc_weight, fc_bias):
    q = (
        tensor.astype(jnp.int32)
        .reshape(_BT, 128, _LT, 8)
        .transpose(2, 0, 3, 1)
    )
    embT = jnp.zeros((_D, 16), jnp.float32).at[:, :_V].set(
        embedding_weight.astype(jnp.float32).T
    )
    params = jnp.concatenate(
        [
            embT.reshape(-1),
            fc_weight.astype(jnp.float32).reshape(-1),
            fc_bias.astype(jnp.float32),
            jnp.zeros((12,), jnp.float32),
        ]
    )
    p = _sc_lookup(q, params)
    return p.transpose(1, 3, 0, 2).reshape(_B, _L, _O)
